# baseline (speedup 1.0000x reference)
import functools

import jax
import jax.numpy as jnp
from jax.experimental import pallas as pl
from jax.experimental.pallas import tpu as pltpu

_THR = 0.5
_BA = 2048


def _assign_body(af_ref, gt_ref, miou_ref, marg_ref, besta_ref,
                 gval_ref, gidx_ref, *, num_anchors, nblk, ba, n_gt):
    blk = pl.program_id(1)

    ax1 = af_ref[0:1, :]
    ay1 = af_ref[1:2, :]
    ax2 = af_ref[2:3, :]
    ay2 = af_ref[3:4, :]

    gt = gt_ref[0]
    gx1 = gt[:, 1:2]
    gy1 = gt[:, 2:3]
    gx2 = gt[:, 3:4]
    gy2 = gt[:, 4:5]

    iw = jnp.maximum(jnp.minimum(gx2, ax2) - jnp.maximum(gx1, ax1), 0.0)
    ih = jnp.maximum(jnp.minimum(gy2, ay2) - jnp.maximum(gy1, ay1), 0.0)
    inter = iw * ih
    area_a = (ax2 - ax1) * (ay2 - ay1)
    area_g = (gx2 - gx1) * (gy2 - gy1)
    union = area_a + area_g - inter
    iou = inter / jnp.maximum(union, 1e-9)

    a_iota = jax.lax.broadcasted_iota(jnp.int32, (1, ba), 1).astype(jnp.float32)
    aidx = a_iota + jnp.float32(ba) * blk.astype(jnp.float32)
    valid = aidx < jnp.float32(num_anchors)
    iou = jnp.where(valid, iou, -1.0)

    amax = jnp.max(iou, axis=0, keepdims=True)
    g_iota = jax.lax.broadcasted_iota(jnp.int32, (n_gt, 1), 0).astype(jnp.float32)
    aarg = jnp.min(jnp.where(iou == amax, g_iota, jnp.float32(n_gt)),
                   axis=0, keepdims=True)
    miou_ref[0] = amax
    marg_ref[0] = aarg

    bmax = jnp.max(iou, axis=1, keepdims=True)
    barg = jnp.min(jnp.where(iou == bmax, aidx, jnp.float32(num_anchors)),
                   axis=1, keepdims=True)

    @pl.when(blk == 0)
    def _init():
        gval_ref[...] = bmax
        gidx_ref[...] = barg

    @pl.when(blk > 0)
    def _merge():
        upd = bmax > gval_ref[...]
        gval_ref[...] = jnp.where(upd, bmax, gval_ref[...])
        gidx_ref[...] = jnp.where(upd, barg, gidx_ref[...])

    @pl.when(blk == nblk - 1)
    def _emit():
        besta_ref[0] = gidx_ref[...]


def _loss_body(af_ref, gtt_ref, cls_ref, reg_ref, miou_ref, marg_ref,
               besta_ref, ocls_ref, oreg_ref, onp_ref,
               *, num_anchors, num_classes, ba, n_gt):
    b = pl.program_id(0)
    blk = pl.program_id(1)

    @pl.when(jnp.logical_and(b == 0, blk == 0))
    def _zero():
        ocls_ref[...] = jnp.zeros((1, 1), jnp.float32)
        oreg_ref[...] = jnp.zeros((1, 1), jnp.float32)
        onp_ref[...] = jnp.zeros((1, 1), jnp.float32)

    a_iota = jax.lax.broadcasted_iota(jnp.int32, (1, ba), 1).astype(jnp.float32)
    aidx = a_iota + jnp.float32(ba) * blk.astype(jnp.float32)
    valid = aidx < jnp.float32(num_anchors)

    g_iota = jax.lax.broadcasted_iota(jnp.int32, (n_gt, 1), 0).astype(jnp.float32)
    miou = miou_ref[0]
    marg = marg_ref[0]
    best = besta_ref[0]

    eq = best == aidx
    has_ovr = jnp.max(jnp.where(eq, 1.0, 0.0), axis=0, keepdims=True) > 0.0
    g_last = jnp.max(jnp.where(eq, g_iota, -1.0), axis=0, keepdims=True)
    m_final = jnp.where(has_ovr, g_last, marg)

    oh = jnp.where(m_final == g_iota, 1.0, 0.0)
    gathered = jax.lax.dot_general(
        gtt_ref[0], oh, (((1,), (0,)), ((), ())),
        preferred_element_type=jnp.float32)
    cls_g = gathered[0:1, :]
    mx1 = gathered[1:2, :]
    my1 = gathered[2:3, :]
    mx2 = gathered[3:4, :]
    my2 = gathered[4:5, :]

    pos_lab = jnp.logical_or(has_ovr, miou >= _THR)
    pos = jnp.logical_and(pos_lab, valid)
    onp_ref[...] += jnp.sum(jnp.where(pos, 1.0, 0.0)).reshape(1, 1)

    acx = af_ref[4:5, :]
    acy = af_ref[5:6, :]
    aw = af_ref[6:7, :]
    ah = af_ref[7:8, :]
    gw = mx2 - mx1
    gh = my2 - my1
    tx = (mx1 + gw * 0.5 - acx) / aw
    ty = (my1 + gh * 0.5 - acy) / ah
    tw = jnp.log(gw / aw)
    th = jnp.log(gh / ah)
    tgt = jnp.concatenate([tx, ty, tw, th], axis=0)
    d = reg_ref[0] - tgt
    ad = jnp.abs(d)
    sl1 = jnp.where(ad < 1.0, 0.5 * d * d, ad - 0.5)
    oreg_ref[...] += jnp.sum(jnp.where(pos, sl1, 0.0)).reshape(1, 1)

    l = jnp.transpose(cls_ref[0], (1, 0))
    c_iota = jax.lax.broadcasted_iota(jnp.int32, (num_classes, 1),
                                      0).astype(jnp.float32)
    onehot = jnp.logical_and(pos_lab, c_iota == cls_g)
    lpos = l >= 0.0
    e = jnp.exp(-jnp.abs(l))
    lp = jnp.log1p(e)
    r = 1.0 / (1.0 + e)
    ce = jnp.maximum(l, 0.0) + lp - jnp.where(onehot, l, 0.0)
    q = jnp.where(onehot == lpos, e, 1.0) * r
    fl = ce * q * q * jnp.where(onehot, 0.25, 0.75)
    fl = jnp.where(valid, fl, 0.0)
    ocls_ref[...] += jnp.sum(fl).reshape(1, 1)


def kernel(anchors, cls_preds, reg_preds, targets):
    A = anchors.shape[0]
    B, _, C = cls_preds.shape
    N = targets.shape[1]
    ba = _BA
    nblk = -(-A // ba)
    a_pad = nblk * ba

    w = anchors[:, 2] - anchors[:, 0]
    h = anchors[:, 3] - anchors[:, 1]
    cxcywh = jnp.stack([anchors[:, 0] + w * 0.5, anchors[:, 1] + h * 0.5,
                        w, h], axis=1)
    af = jnp.concatenate([anchors, cxcywh], axis=1)
    af = jnp.pad(af, ((0, a_pad - A), (0, 0)), constant_values=1.0).T

    gt = jnp.pad(targets, ((0, 0), (0, 0), (0, 3)))
    gtt = jnp.transpose(gt, (0, 2, 1))

    reg_t = jnp.transpose(reg_preds, (0, 2, 1))

    grid = (B, nblk)
    af_spec = pl.BlockSpec((8, ba), lambda b, k: (0, k))
    gt_spec = pl.BlockSpec((1, N, 8), lambda b, k: (b, 0, 0))
    gtt_spec = pl.BlockSpec((1, 8, N), lambda b, k: (b, 0, 0))
    row_spec = pl.BlockSpec((1, 1, ba), lambda b, k: (b, 0, k))
    besta_spec = pl.BlockSpec((1, N, 1), lambda b, k: (b, 0, 0))

    miou, marg, besta = pl.pallas_call(
        functools.partial(_assign_body, num_anchors=A, nblk=nblk, ba=ba,
                          n_gt=N),
        grid=grid,
        in_specs=[af_spec, gt_spec],
        out_specs=[row_spec, row_spec, besta_spec],
        out_shape=[
            jax.ShapeDtypeStruct((B, 1, a_pad), jnp.float32),
            jax.ShapeDtypeStruct((B, 1, a_pad), jnp.float32),
            jax.ShapeDtypeStruct((B, N, 1), jnp.float32),
        ],
        scratch_shapes=[
            pltpu.VMEM((N, 1), jnp.float32),
            pltpu.VMEM((N, 1), jnp.float32),
        ],
    )(af, gt)

    scal_spec = pl.BlockSpec((1, 1), lambda b, k: (0, 0))
    s_cls, s_reg, s_np = pl.pallas_call(
        functools.partial(_loss_body, num_anchors=A, num_classes=C, ba=ba,
                          n_gt=N),
        grid=grid,
        in_specs=[
            af_spec,
            gtt_spec,
            pl.BlockSpec((1, ba, C), lambda b, k: (b, k, 0)),
            pl.BlockSpec((1, 4, ba), lambda b, k: (b, 0, k)),
            row_spec,
            row_spec,
            besta_spec,
        ],
        out_specs=[scal_spec, scal_spec, scal_spec],
        out_shape=[
            jax.ShapeDtypeStruct((1, 1), jnp.float32),
            jax.ShapeDtypeStruct((1, 1), jnp.float32),
            jax.ShapeDtypeStruct((1, 1), jnp.float32),
        ],
    )(af, gtt, cls_preds, reg_t, miou, marg, besta)

    num_pos = jnp.maximum(s_np[0, 0], 1.0)
    return s_cls[0, 0] / num_pos, s_reg[0, 0] / num_pos


# speedup vs baseline: 1.2060x; 1.2060x over previous
import functools

import jax
import jax.numpy as jnp
from jax.experimental import pallas as pl
from jax.experimental.pallas import tpu as pltpu

_THR = 0.5
_BA = 2048


def _assign_body(af_ref, gt_ref, miou_ref, marg_ref, besta_ref,
                 gval_ref, gidx_ref, *, num_anchors, nblk, ba, n_gt):
    blk = pl.program_id(1)

    ax1 = af_ref[0:1, :]
    ay1 = af_ref[1:2, :]
    ax2 = af_ref[2:3, :]
    ay2 = af_ref[3:4, :]

    gt = gt_ref[0]
    gx1 = gt[:, 1:2]
    gy1 = gt[:, 2:3]
    gx2 = gt[:, 3:4]
    gy2 = gt[:, 4:5]

    iw = jnp.maximum(jnp.minimum(gx2, ax2) - jnp.maximum(gx1, ax1), 0.0)
    ih = jnp.maximum(jnp.minimum(gy2, ay2) - jnp.maximum(gy1, ay1), 0.0)
    inter = iw * ih
    area_a = (ax2 - ax1) * (ay2 - ay1)
    area_g = (gx2 - gx1) * (gy2 - gy1)
    union = area_a + area_g - inter
    iou = inter / jnp.maximum(union, 1e-9)

    a_iota = jax.lax.broadcasted_iota(jnp.int32, (1, ba), 1).astype(jnp.float32)
    aidx = a_iota + jnp.float32(ba) * blk.astype(jnp.float32)
    valid = aidx < jnp.float32(num_anchors)
    iou = jnp.where(valid, iou, -1.0)

    amax = jnp.max(iou, axis=0, keepdims=True)
    g_iota = jax.lax.broadcasted_iota(jnp.int32, (n_gt, 1), 0).astype(jnp.float32)
    aarg = jnp.min(jnp.where(iou == amax, g_iota, jnp.float32(n_gt)),
                   axis=0, keepdims=True)
    miou_ref[0] = amax
    marg_ref[0] = aarg

    bmax = jnp.max(iou, axis=1, keepdims=True)
    barg = jnp.min(jnp.where(iou == bmax, aidx, jnp.float32(num_anchors)),
                   axis=1, keepdims=True)

    @pl.when(blk == 0)
    def _init():
        gval_ref[...] = bmax
        gidx_ref[...] = barg

    @pl.when(blk > 0)
    def _merge():
        upd = bmax > gval_ref[...]
        gval_ref[...] = jnp.where(upd, bmax, gval_ref[...])
        gidx_ref[...] = jnp.where(upd, barg, gidx_ref[...])

    @pl.when(blk == nblk - 1)
    def _emit():
        besta_ref[0] = gidx_ref[...]


def _loss_body(af_ref, gtt_ref, cls_ref, reg_ref, miou_ref, marg_ref,
               besta_ref, ocls_ref, oreg_ref, onp_ref,
               *, num_anchors, num_classes, ba, n_gt, nblk):
    b = pl.program_id(0)
    blk = pl.program_id(1)

    @pl.when(jnp.logical_and(b == 0, blk == 0))
    def _zero():
        ocls_ref[...] = jnp.zeros((1, 1), jnp.float32)
        oreg_ref[...] = jnp.zeros((1, 1), jnp.float32)
        onp_ref[...] = jnp.zeros((1, 1), jnp.float32)

    a_iota = jax.lax.broadcasted_iota(jnp.int32, (1, ba), 1).astype(jnp.float32)
    aidx = a_iota + jnp.float32(ba) * blk.astype(jnp.float32)
    valid = aidx < jnp.float32(num_anchors)

    g_iota = jax.lax.broadcasted_iota(jnp.int32, (n_gt, 1), 0).astype(jnp.float32)
    miou = miou_ref[0]
    marg = marg_ref[0]
    best = besta_ref[0]

    eq = best == aidx
    has_ovr = jnp.max(jnp.where(eq, 1.0, 0.0), axis=0, keepdims=True) > 0.0
    g_last = jnp.max(jnp.where(eq, g_iota, -1.0), axis=0, keepdims=True)
    m_final = jnp.where(has_ovr, g_last, marg)

    oh = jnp.where(m_final == g_iota, 1.0, 0.0)
    gathered = jax.lax.dot_general(
        gtt_ref[0], oh, (((1,), (0,)), ((), ())),
        preferred_element_type=jnp.float32)
    cls_g = gathered[0:1, :]
    mx1 = gathered[1:2, :]
    my1 = gathered[2:3, :]
    mx2 = gathered[3:4, :]
    my2 = gathered[4:5, :]

    pos_lab = jnp.logical_or(has_ovr, miou >= _THR)
    pos = jnp.logical_and(pos_lab, valid)
    onp_ref[...] += jnp.sum(jnp.where(pos, 1.0, 0.0)).reshape(1, 1)

    acx = af_ref[4:5, :]
    acy = af_ref[5:6, :]
    aw = af_ref[6:7, :]
    ah = af_ref[7:8, :]
    gw = mx2 - mx1
    gh = my2 - my1
    tx = (mx1 + gw * 0.5 - acx) / aw
    ty = (my1 + gh * 0.5 - acy) / ah
    tw = jnp.log(gw / aw)
    th = jnp.log(gh / ah)
    tgt = jnp.concatenate([tx, ty, tw, th], axis=0)
    d = reg_ref[0] - tgt
    ad = jnp.abs(d)
    sl1 = jnp.where(ad < 1.0, 0.5 * d * d, ad - 0.5)
    oreg_ref[...] += jnp.sum(jnp.where(pos, sl1, 0.0)).reshape(1, 1)

    l = cls_ref[0]
    c_iota = jax.lax.broadcasted_iota(jnp.int32, (num_classes, 1),
                                      0).astype(jnp.float32)
    onehot = jnp.logical_and(pos_lab, c_iota == cls_g)
    lpos = l >= 0.0
    e = jnp.exp(-jnp.abs(l))
    lp = jnp.log1p(e)
    r = 1.0 / (1.0 + e)
    ce = jnp.maximum(l, 0.0) + lp - jnp.where(onehot, l, 0.0)
    q = jnp.where(onehot == lpos, e, 1.0) * r
    fl = ce * q * q * jnp.where(onehot, 0.25, 0.75)

    @pl.when(blk < nblk - 1)
    def _acc_full():
        ocls_ref[...] += jnp.sum(fl).reshape(1, 1)

    @pl.when(blk == nblk - 1)
    def _acc_masked():
        ocls_ref[...] += jnp.sum(jnp.where(valid, fl, 0.0)).reshape(1, 1)


def kernel(anchors, cls_preds, reg_preds, targets):
    A = anchors.shape[0]
    B, _, C = cls_preds.shape
    N = targets.shape[1]
    ba = _BA
    nblk = -(-A // ba)
    a_pad = nblk * ba

    w = anchors[:, 2] - anchors[:, 0]
    h = anchors[:, 3] - anchors[:, 1]
    cxcywh = jnp.stack([anchors[:, 0] + w * 0.5, anchors[:, 1] + h * 0.5,
                        w, h], axis=1)
    af = jnp.concatenate([anchors, cxcywh], axis=1)
    af = jnp.pad(af, ((0, a_pad - A), (0, 0)), constant_values=1.0).T

    gt = jnp.pad(targets, ((0, 0), (0, 0), (0, 3)))
    gtt = jnp.transpose(gt, (0, 2, 1))

    cls_t = jnp.transpose(cls_preds, (0, 2, 1))
    reg_t = jnp.transpose(reg_preds, (0, 2, 1))

    grid = (B, nblk)
    af_spec = pl.BlockSpec((8, ba), lambda b, k: (0, k))
    gt_spec = pl.BlockSpec((1, N, 8), lambda b, k: (b, 0, 0))
    gtt_spec = pl.BlockSpec((1, 8, N), lambda b, k: (b, 0, 0))
    row_spec = pl.BlockSpec((1, 1, ba), lambda b, k: (b, 0, k))
    besta_spec = pl.BlockSpec((1, N, 1), lambda b, k: (b, 0, 0))

    miou, marg, besta = pl.pallas_call(
        functools.partial(_assign_body, num_anchors=A, nblk=nblk, ba=ba,
                          n_gt=N),
        grid=grid,
        in_specs=[af_spec, gt_spec],
        out_specs=[row_spec, row_spec, besta_spec],
        out_shape=[
            jax.ShapeDtypeStruct((B, 1, a_pad), jnp.float32),
            jax.ShapeDtypeStruct((B, 1, a_pad), jnp.float32),
            jax.ShapeDtypeStruct((B, N, 1), jnp.float32),
        ],
        scratch_shapes=[
            pltpu.VMEM((N, 1), jnp.float32),
            pltpu.VMEM((N, 1), jnp.float32),
        ],
    )(af, gt)

    scal_spec = pl.BlockSpec((1, 1), lambda b, k: (0, 0))
    s_cls, s_reg, s_np = pl.pallas_call(
        functools.partial(_loss_body, num_anchors=A, num_classes=C, ba=ba,
                          n_gt=N, nblk=nblk),
        grid=grid,
        in_specs=[
            af_spec,
            gtt_spec,
            pl.BlockSpec((1, C, ba), lambda b, k: (b, 0, k)),
            pl.BlockSpec((1, 4, ba), lambda b, k: (b, 0, k)),
            row_spec,
            row_spec,
            besta_spec,
        ],
        out_specs=[scal_spec, scal_spec, scal_spec],
        out_shape=[
            jax.ShapeDtypeStruct((1, 1), jnp.float32),
            jax.ShapeDtypeStruct((1, 1), jnp.float32),
            jax.ShapeDtypeStruct((1, 1), jnp.float32),
        ],
    )(af, gtt, cls_t, reg_t, miou, marg, besta)

    num_pos = jnp.maximum(s_np[0, 0], 1.0)
    return s_cls[0, 0] / num_pos, s_reg[0, 0] / num_pos


# speedup vs baseline: 1.2953x; 1.0740x over previous
import functools

import jax
import jax.numpy as jnp
from jax.experimental import pallas as pl
from jax.experimental.pallas import tpu as pltpu

_THR = 0.5
_BA = 2048


def _assign_body(af_ref, gt_ref, miou_ref, marg_ref, besta_ref,
                 gval_ref, gidx_ref, *, num_anchors, nblk, ba, n_gt):
    blk = pl.program_id(1)

    ax1 = af_ref[0:1, :]
    ay1 = af_ref[1:2, :]
    ax2 = af_ref[2:3, :]
    ay2 = af_ref[3:4, :]

    gt = gt_ref[0]
    gx1 = gt[:, 1:2]
    gy1 = gt[:, 2:3]
    gx2 = gt[:, 3:4]
    gy2 = gt[:, 4:5]

    iw = jnp.maximum(jnp.minimum(gx2, ax2) - jnp.maximum(gx1, ax1), 0.0)
    ih = jnp.maximum(jnp.minimum(gy2, ay2) - jnp.maximum(gy1, ay1), 0.0)
    inter = iw * ih
    area_a = (ax2 - ax1) * (ay2 - ay1)
    area_g = (gx2 - gx1) * (gy2 - gy1)
    union = area_a + area_g - inter
    iou = inter / jnp.maximum(union, 1e-9)

    a_iota = jax.lax.broadcasted_iota(jnp.int32, (1, ba), 1).astype(jnp.float32)
    aidx = a_iota + jnp.float32(ba) * blk.astype(jnp.float32)
    valid = aidx < jnp.float32(num_anchors)
    iou = jnp.where(valid, iou, -1.0)

    amax = jnp.max(iou, axis=0, keepdims=True)
    g_iota = jax.lax.broadcasted_iota(jnp.int32, (n_gt, 1), 0).astype(jnp.float32)
    aarg = jnp.min(jnp.where(iou == amax, g_iota, jnp.float32(n_gt)),
                   axis=0, keepdims=True)
    miou_ref[0] = amax
    marg_ref[0] = aarg

    bmax = jnp.max(iou, axis=1, keepdims=True)
    barg = jnp.min(jnp.where(iou == bmax, aidx, jnp.float32(num_anchors)),
                   axis=1, keepdims=True)

    @pl.when(blk == 0)
    def _init():
        gval_ref[...] = bmax
        gidx_ref[...] = barg

    @pl.when(blk > 0)
    def _merge():
        upd = bmax > gval_ref[...]
        gval_ref[...] = jnp.where(upd, bmax, gval_ref[...])
        gidx_ref[...] = jnp.where(upd, barg, gidx_ref[...])

    @pl.when(blk == nblk - 1)
    def _emit():
        besta_ref[0] = gidx_ref[...]


def _loss_body(af_ref, gtt_ref, cls_ref, reg_ref, miou_ref, marg_ref,
               besta_ref, ocls_ref, oreg_ref, onp_ref,
               *, num_anchors, num_classes, ba, n_gt):
    b = pl.program_id(0)
    blk = pl.program_id(1)

    @pl.when(jnp.logical_and(b == 0, blk == 0))
    def _zero():
        ocls_ref[...] = jnp.zeros((1, 1), jnp.float32)
        oreg_ref[...] = jnp.zeros((1, 1), jnp.float32)
        onp_ref[...] = jnp.zeros((1, 1), jnp.float32)

    a_iota = jax.lax.broadcasted_iota(jnp.int32, (1, ba), 1).astype(jnp.float32)
    aidx = a_iota + jnp.float32(ba) * blk.astype(jnp.float32)
    valid = aidx < jnp.float32(num_anchors)

    g_iota = jax.lax.broadcasted_iota(jnp.int32, (n_gt, 1), 0).astype(jnp.float32)
    miou = miou_ref[0]
    marg = marg_ref[0]
    best = besta_ref[0]

    eq = best == aidx
    has_ovr = jnp.max(jnp.where(eq, 1.0, 0.0), axis=0, keepdims=True) > 0.0
    g_last = jnp.max(jnp.where(eq, g_iota, -1.0), axis=0, keepdims=True)
    m_final = jnp.where(has_ovr, g_last, marg)

    oh = jnp.where(m_final == g_iota, 1.0, 0.0)
    gathered = jax.lax.dot_general(
        gtt_ref[0], oh, (((1,), (0,)), ((), ())),
        preferred_element_type=jnp.float32)
    cls_g = gathered[0:1, :]
    mx1 = gathered[1:2, :]
    my1 = gathered[2:3, :]
    mx2 = gathered[3:4, :]
    my2 = gathered[4:5, :]

    pos_lab = jnp.logical_or(has_ovr, miou >= _THR)
    pos = jnp.logical_and(pos_lab, valid)
    onp_ref[...] += jnp.sum(jnp.where(pos, 1.0, 0.0)).reshape(1, 1)

    acx = af_ref[4:5, :]
    acy = af_ref[5:6, :]
    aw = af_ref[6:7, :]
    ah = af_ref[7:8, :]
    gw = mx2 - mx1
    gh = my2 - my1
    tx = (mx1 + gw * 0.5 - acx) / aw
    ty = (my1 + gh * 0.5 - acy) / ah
    tw = jnp.log(gw / aw)
    th = jnp.log(gh / ah)
    tgt = jnp.concatenate([tx, ty, tw, th], axis=0)
    d = reg_ref[0] - tgt
    ad = jnp.abs(d)
    sl1 = jnp.where(ad < 1.0, 0.5 * d * d, ad - 0.5)
    oreg_ref[...] += jnp.sum(jnp.where(pos, sl1, 0.0)).reshape(1, 1)

    l = cls_ref[0].astype(jnp.float32)
    c_iota = jax.lax.broadcasted_iota(jnp.int32, (num_classes, 1),
                                      0).astype(jnp.float32)
    onehot = jnp.logical_and(pos_lab, c_iota == cls_g)
    p = 0.5 + 0.5 * jnp.tanh(l * 0.5)
    pt = jnp.where(onehot, p, 1.0 - p)
    pt = jnp.maximum(pt, 1e-30)
    ce = -jnp.log(pt)
    q = 1.0 - pt
    fl = ce * q * q * jnp.where(onehot, 0.25, 0.75)
    fl = jnp.where(valid, fl, 0.0)
    ocls_ref[...] += jnp.sum(fl).reshape(1, 1)


def kernel(anchors, cls_preds, reg_preds, targets):
    A = anchors.shape[0]
    B, _, C = cls_preds.shape
    N = targets.shape[1]
    ba = _BA
    nblk = -(-A // ba)
    a_pad = nblk * ba

    w = anchors[:, 2] - anchors[:, 0]
    h = anchors[:, 3] - anchors[:, 1]
    cxcywh = jnp.stack([anchors[:, 0] + w * 0.5, anchors[:, 1] + h * 0.5,
                        w, h], axis=1)
    af = jnp.concatenate([anchors, cxcywh], axis=1)
    af = jnp.pad(af, ((0, a_pad - A), (0, 0)), constant_values=1.0).T

    gt = jnp.pad(targets, ((0, 0), (0, 0), (0, 3)))
    gtt = jnp.transpose(gt, (0, 2, 1))

    cls_t = jnp.transpose(cls_preds.astype(jnp.bfloat16), (0, 2, 1))
    reg_t = jnp.transpose(reg_preds, (0, 2, 1))

    grid = (B, nblk)
    af_spec = pl.BlockSpec((8, ba), lambda b, k: (0, k))
    gt_spec = pl.BlockSpec((1, N, 8), lambda b, k: (b, 0, 0))
    gtt_spec = pl.BlockSpec((1, 8, N), lambda b, k: (b, 0, 0))
    row_spec = pl.BlockSpec((1, 1, ba), lambda b, k: (b, 0, k))
    besta_spec = pl.BlockSpec((1, N, 1), lambda b, k: (b, 0, 0))

    miou, marg, besta = pl.pallas_call(
        functools.partial(_assign_body, num_anchors=A, nblk=nblk, ba=ba,
                          n_gt=N),
        grid=grid,
        in_specs=[af_spec, gt_spec],
        out_specs=[row_spec, row_spec, besta_spec],
        out_shape=[
            jax.ShapeDtypeStruct((B, 1, a_pad), jnp.float32),
            jax.ShapeDtypeStruct((B, 1, a_pad), jnp.float32),
            jax.ShapeDtypeStruct((B, N, 1), jnp.float32),
        ],
        scratch_shapes=[
            pltpu.VMEM((N, 1), jnp.float32),
            pltpu.VMEM((N, 1), jnp.float32),
        ],
    )(af, gt)

    scal_spec = pl.BlockSpec((1, 1), lambda b, k: (0, 0))
    s_cls, s_reg, s_np = pl.pallas_call(
        functools.partial(_loss_body, num_anchors=A, num_classes=C, ba=ba,
                          n_gt=N),
        grid=grid,
        in_specs=[
            af_spec,
            gtt_spec,
            pl.BlockSpec((1, C, ba), lambda b, k: (b, 0, k)),
            pl.BlockSpec((1, 4, ba), lambda b, k: (b, 0, k)),
            row_spec,
            row_spec,
            besta_spec,
        ],
        out_specs=[scal_spec, scal_spec, scal_spec],
        out_shape=[
            jax.ShapeDtypeStruct((1, 1), jnp.float32),
            jax.ShapeDtypeStruct((1, 1), jnp.float32),
            jax.ShapeDtypeStruct((1, 1), jnp.float32),
        ],
    )(af, gtt, cls_t, reg_t, miou, marg, besta)

    num_pos = jnp.maximum(s_np[0, 0], 1.0)
    return s_cls[0, 0] / num_pos, s_reg[0, 0] / num_pos


# speedup vs baseline: 1.4704x; 1.1352x over previous
import functools

import jax
import jax.numpy as jnp
from jax.experimental import pallas as pl
from jax.experimental.pallas import tpu as pltpu

_THR = 0.5
_BA = 2048


def _body(af_ref, gt_ref, gtt_ref, cls_ref, reg_ref,
          ocls_ref, oreg_ref, onp_ref,
          miou_ref, marg_ref, gval_ref, gidx_ref,
          *, num_anchors, num_classes, nblk, ba, n_gt):
    b = pl.program_id(0)
    p = pl.program_id(1)
    blk = pl.program_id(2)

    a_iota = jax.lax.broadcasted_iota(jnp.int32, (1, ba), 1).astype(jnp.float32)
    aidx = a_iota + jnp.float32(ba) * blk.astype(jnp.float32)
    valid = aidx < jnp.float32(num_anchors)
    g_iota = jax.lax.broadcasted_iota(jnp.int32, (n_gt, 1), 0).astype(jnp.float32)

    @pl.when(p == 0)
    def _assign():
        ax1 = af_ref[0:1, :]
        ay1 = af_ref[1:2, :]
        ax2 = af_ref[2:3, :]
        ay2 = af_ref[3:4, :]
        gt = gt_ref[0]
        gx1 = gt[:, 1:2]
        gy1 = gt[:, 2:3]
        gx2 = gt[:, 3:4]
        gy2 = gt[:, 4:5]

        iw = jnp.maximum(jnp.minimum(gx2, ax2) - jnp.maximum(gx1, ax1), 0.0)
        ih = jnp.maximum(jnp.minimum(gy2, ay2) - jnp.maximum(gy1, ay1), 0.0)
        inter = iw * ih
        area_a = (ax2 - ax1) * (ay2 - ay1)
        area_g = (gx2 - gx1) * (gy2 - gy1)
        union = area_a + area_g - inter
        iou = inter / jnp.maximum(union, 1e-9)
        iou = jnp.where(valid, iou, -1.0)

        amax = jnp.max(iou, axis=0, keepdims=True)
        aarg = jnp.min(jnp.where(iou == amax, g_iota, jnp.float32(n_gt)),
                       axis=0, keepdims=True)
        miou_ref[0:1, pl.ds(blk * ba, ba)] = amax
        marg_ref[0:1, pl.ds(blk * ba, ba)] = aarg

        bmax = jnp.max(iou, axis=1, keepdims=True)
        barg = jnp.min(jnp.where(iou == bmax, aidx, jnp.float32(num_anchors)),
                       axis=1, keepdims=True)

        @pl.when(blk == 0)
        def _init():
            gval_ref[...] = bmax
            gidx_ref[...] = barg

        @pl.when(blk > 0)
        def _merge():
            upd = bmax > gval_ref[...]
            gval_ref[...] = jnp.where(upd, bmax, gval_ref[...])
            gidx_ref[...] = jnp.where(upd, barg, gidx_ref[...])

    @pl.when(p == 1)
    def _loss():
        @pl.when(jnp.logical_and(b == 0, blk == 0))
        def _zero():
            ocls_ref[...] = jnp.zeros((1, 1), jnp.float32)
            oreg_ref[...] = jnp.zeros((1, 1), jnp.float32)
            onp_ref[...] = jnp.zeros((1, 1), jnp.float32)

        miou = miou_ref[0:1, pl.ds(blk * ba, ba)]
        marg = marg_ref[0:1, pl.ds(blk * ba, ba)]
        best = gidx_ref[...]

        eq = best == aidx
        has_ovr = jnp.max(jnp.where(eq, 1.0, 0.0), axis=0, keepdims=True) > 0.0
        g_last = jnp.max(jnp.where(eq, g_iota, -1.0), axis=0, keepdims=True)
        m_final = jnp.where(has_ovr, g_last, marg)

        oh = jnp.where(m_final == g_iota, 1.0, 0.0)
        gathered = jax.lax.dot_general(
            gtt_ref[0], oh, (((1,), (0,)), ((), ())),
            preferred_element_type=jnp.float32)
        cls_g = gathered[0:1, :]
        mx1 = gathered[1:2, :]
        my1 = gathered[2:3, :]
        mx2 = gathered[3:4, :]
        my2 = gathered[4:5, :]

        pos_lab = jnp.logical_or(has_ovr, miou >= _THR)
        pos = jnp.logical_and(pos_lab, valid)
        onp_ref[...] += jnp.sum(jnp.where(pos, 1.0, 0.0)).reshape(1, 1)

        acx = af_ref[4:5, :]
        acy = af_ref[5:6, :]
        aw = af_ref[6:7, :]
        ah = af_ref[7:8, :]
        gw = mx2 - mx1
        gh = my2 - my1
        tx = (mx1 + gw * 0.5 - acx) / aw
        ty = (my1 + gh * 0.5 - acy) / ah
        tw = jnp.log(gw / aw)
        th = jnp.log(gh / ah)
        tgt = jnp.concatenate([tx, ty, tw, th], axis=0)
        d = reg_ref[0] - tgt
        ad = jnp.abs(d)
        sl1 = jnp.where(ad < 1.0, 0.5 * d * d, ad - 0.5)
        oreg_ref[...] += jnp.sum(jnp.where(pos, sl1, 0.0)).reshape(1, 1)

        l = cls_ref[0]
        c_iota = jax.lax.broadcasted_iota(jnp.int32, (num_classes, 1),
                                          0).astype(jnp.float32)
        onehot = jnp.logical_and(pos_lab, c_iota == cls_g)
        pcls = 0.5 + 0.5 * jnp.tanh(l * 0.5)
        pt = jnp.where(onehot, pcls, 1.0 - pcls)
        pt = jnp.maximum(pt, 1e-30)
        ce = -jnp.log(pt)
        q = 1.0 - pt
        fl = ce * q * q * jnp.where(onehot, 0.25, 0.75)
        fl = jnp.where(valid, fl, 0.0)
        ocls_ref[...] += jnp.sum(fl).reshape(1, 1)


def kernel(anchors, cls_preds, reg_preds, targets):
    A = anchors.shape[0]
    B, _, C = cls_preds.shape
    N = targets.shape[1]
    ba = _BA
    nblk = -(-A // ba)
    a_pad = nblk * ba

    w = anchors[:, 2] - anchors[:, 0]
    h = anchors[:, 3] - anchors[:, 1]
    cxcywh = jnp.stack([anchors[:, 0] + w * 0.5, anchors[:, 1] + h * 0.5,
                        w, h], axis=1)
    af = jnp.concatenate([anchors, cxcywh], axis=1)
    af = jnp.pad(af, ((0, a_pad - A), (0, 0)), constant_values=1.0).T

    gt = jnp.pad(targets, ((0, 0), (0, 0), (0, 3)))
    gtt = jnp.transpose(gt, (0, 2, 1))
    cls_t = jnp.transpose(cls_preds, (0, 2, 1))
    reg_t = jnp.transpose(reg_preds, (0, 2, 1))

    grid = (B, 2, nblk)
    scal_spec = pl.BlockSpec((1, 1), lambda b, p, k: (0, 0))
    s_cls, s_reg, s_np = pl.pallas_call(
        functools.partial(_body, num_anchors=A, num_classes=C, nblk=nblk,
                          ba=ba, n_gt=N),
        grid=grid,
        in_specs=[
            pl.BlockSpec((8, ba), lambda b, p, k: (0, k)),
            pl.BlockSpec((1, N, 8), lambda b, p, k: (b, 0, 0)),
            pl.BlockSpec((1, 8, N), lambda b, p, k: (b, 0, 0)),
            pl.BlockSpec((1, C, ba), lambda b, p, k: (b, 0, k * p)),
            pl.BlockSpec((1, 4, ba), lambda b, p, k: (b, 0, k * p)),
        ],
        out_specs=[scal_spec, scal_spec, scal_spec],
        out_shape=[
            jax.ShapeDtypeStruct((1, 1), jnp.float32),
            jax.ShapeDtypeStruct((1, 1), jnp.float32),
            jax.ShapeDtypeStruct((1, 1), jnp.float32),
        ],
        scratch_shapes=[
            pltpu.VMEM((1, a_pad), jnp.float32),
            pltpu.VMEM((1, a_pad), jnp.float32),
            pltpu.VMEM((N, 1), jnp.float32),
            pltpu.VMEM((N, 1), jnp.float32),
        ],
    )(af, gt, gtt, cls_t, reg_t)

    num_pos = jnp.maximum(s_np[0, 0], 1.0)
    return s_cls[0, 0] / num_pos, s_reg[0, 0] / num_pos


# speedup vs baseline: 1.4758x; 1.0036x over previous
import functools

import jax
import jax.numpy as jnp
from jax.experimental import pallas as pl
from jax.experimental.pallas import tpu as pltpu

_THR = 0.5
_BA = 2048


def _assign_body(af_ref, gt_ref, miou_ref, marg_ref, besta_ref,
                 gval_ref, gidx_ref, *, num_anchors, nblk, ba, n_gt):
    blk = pl.program_id(1)

    ax1 = af_ref[0:1, :]
    ay1 = af_ref[1:2, :]
    ax2 = af_ref[2:3, :]
    ay2 = af_ref[3:4, :]

    gt = gt_ref[0]
    gx1 = gt[:, 1:2]
    gy1 = gt[:, 2:3]
    gx2 = gt[:, 3:4]
    gy2 = gt[:, 4:5]

    iw = jnp.maximum(jnp.minimum(gx2, ax2) - jnp.maximum(gx1, ax1), 0.0)
    ih = jnp.maximum(jnp.minimum(gy2, ay2) - jnp.maximum(gy1, ay1), 0.0)
    inter = iw * ih
    area_a = (ax2 - ax1) * (ay2 - ay1)
    area_g = (gx2 - gx1) * (gy2 - gy1)
    union = area_a + area_g - inter
    iou = inter / jnp.maximum(union, 1e-9)

    a_iota = jax.lax.broadcasted_iota(jnp.int32, (1, ba), 1).astype(jnp.float32)
    aidx = a_iota + jnp.float32(ba) * blk.astype(jnp.float32)
    valid = aidx < jnp.float32(num_anchors)
    iou = jnp.where(valid, iou, -1.0)

    amax = jnp.max(iou, axis=0, keepdims=True)
    g_iota = jax.lax.broadcasted_iota(jnp.int32, (n_gt, 1), 0).astype(jnp.float32)
    aarg = jnp.min(jnp.where(iou == amax, g_iota, jnp.float32(n_gt)),
                   axis=0, keepdims=True)
    miou_ref[0] = amax
    marg_ref[0] = aarg

    bmax = jnp.max(iou, axis=1, keepdims=True)
    barg = jnp.min(jnp.where(iou == bmax, aidx, jnp.float32(num_anchors)),
                   axis=1, keepdims=True)

    @pl.when(blk == 0)
    def _init():
        gval_ref[...] = bmax
        gidx_ref[...] = barg

    @pl.when(blk > 0)
    def _merge():
        upd = bmax > gval_ref[...]
        gval_ref[...] = jnp.where(upd, bmax, gval_ref[...])
        gidx_ref[...] = jnp.where(upd, barg, gidx_ref[...])

    @pl.when(blk == nblk - 1)
    def _emit():
        besta_ref[0] = gidx_ref[...]


def _loss_body(af_ref, gtt_ref, cls_ref, reg_ref, miou_ref, marg_ref,
               besta_ref, ocls_ref, oreg_ref, onp_ref,
               *, num_anchors, num_classes, ba, n_gt):
    b = pl.program_id(0)
    blk = pl.program_id(1)

    @pl.when(jnp.logical_and(b == 0, blk == 0))
    def _zero():
        ocls_ref[...] = jnp.zeros((1, 1), jnp.float32)
        oreg_ref[...] = jnp.zeros((1, 1), jnp.float32)
        onp_ref[...] = jnp.zeros((1, 1), jnp.float32)

    a_iota = jax.lax.broadcasted_iota(jnp.int32, (1, ba), 1).astype(jnp.float32)
    aidx = a_iota + jnp.float32(ba) * blk.astype(jnp.float32)
    valid = aidx < jnp.float32(num_anchors)

    g_iota = jax.lax.broadcasted_iota(jnp.int32, (n_gt, 1), 0).astype(jnp.float32)
    miou = miou_ref[0]
    marg = marg_ref[0]
    best = besta_ref[0]

    eq = best == aidx
    has_ovr = jnp.max(jnp.where(eq, 1.0, 0.0), axis=0, keepdims=True) > 0.0
    g_last = jnp.max(jnp.where(eq, g_iota, -1.0), axis=0, keepdims=True)
    m_final = jnp.where(has_ovr, g_last, marg)

    oh = jnp.where(m_final == g_iota, 1.0, 0.0)
    gathered = jax.lax.dot_general(
        gtt_ref[0], oh, (((1,), (0,)), ((), ())),
        preferred_element_type=jnp.float32)
    cls_g = gathered[0:1, :]
    mx1 = gathered[1:2, :]
    my1 = gathered[2:3, :]
    mx2 = gathered[3:4, :]
    my2 = gathered[4:5, :]

    pos_lab = jnp.logical_or(has_ovr, miou >= _THR)
    pos = jnp.logical_and(pos_lab, valid)
    onp_ref[...] += jnp.sum(jnp.where(pos, 1.0, 0.0)).reshape(1, 1)

    acx = af_ref[4:5, :]
    acy = af_ref[5:6, :]
    aw = af_ref[6:7, :]
    ah = af_ref[7:8, :]
    gw = mx2 - mx1
    gh = my2 - my1
    tx = (mx1 + gw * 0.5 - acx) / aw
    ty = (my1 + gh * 0.5 - acy) / ah
    tw = jnp.log(gw / aw)
    th = jnp.log(gh / ah)
    tgt = jnp.concatenate([tx, ty, tw, th], axis=0)
    d = reg_ref[0] - tgt
    ad = jnp.abs(d)
    sl1 = jnp.where(ad < 1.0, 0.5 * d * d, ad - 0.5)
    oreg_ref[...] += jnp.sum(jnp.where(pos, sl1, 0.0)).reshape(1, 1)

    l = cls_ref[0]
    c_iota = jax.lax.broadcasted_iota(jnp.int32, (num_classes, 1),
                                      0).astype(jnp.float32)
    onehot = jnp.logical_and(pos_lab, c_iota == cls_g)
    p = 0.5 + 0.5 * jnp.tanh(l * 0.5)
    pt = jnp.where(onehot, p, 1.0 - p)
    pt = jnp.maximum(pt, 1e-30)
    ce = -jnp.log(pt)
    q = 1.0 - pt
    fl = ce * q * q * jnp.where(onehot, 0.25, 0.75)
    fl = jnp.where(valid, fl, 0.0)
    ocls_ref[...] += jnp.sum(fl).reshape(1, 1)


def kernel(anchors, cls_preds, reg_preds, targets):
    A = anchors.shape[0]
    B, _, C = cls_preds.shape
    N = targets.shape[1]
    ba = _BA
    nblk = -(-A // ba)
    a_pad = nblk * ba

    w = anchors[:, 2] - anchors[:, 0]
    h = anchors[:, 3] - anchors[:, 1]
    cxcywh = jnp.stack([anchors[:, 0] + w * 0.5, anchors[:, 1] + h * 0.5,
                        w, h], axis=1)
    af = jnp.concatenate([anchors, cxcywh], axis=1)
    af = jnp.pad(af, ((0, a_pad - A), (0, 0)), constant_values=1.0).T

    gt = jnp.pad(targets, ((0, 0), (0, 0), (0, 3)))
    gtt = jnp.transpose(gt, (0, 2, 1))

    cls_t = jnp.transpose(cls_preds, (0, 2, 1))
    reg_t = jnp.transpose(reg_preds, (0, 2, 1))

    grid = (B, nblk)
    af_spec = pl.BlockSpec((8, ba), lambda b, k: (0, k))
    gt_spec = pl.BlockSpec((1, N, 8), lambda b, k: (b, 0, 0))
    gtt_spec = pl.BlockSpec((1, 8, N), lambda b, k: (b, 0, 0))
    row_spec = pl.BlockSpec((1, 1, ba), lambda b, k: (b, 0, k))
    besta_spec = pl.BlockSpec((1, N, 1), lambda b, k: (b, 0, 0))

    miou, marg, besta = pl.pallas_call(
        functools.partial(_assign_body, num_anchors=A, nblk=nblk, ba=ba,
                          n_gt=N),
        grid=grid,
        in_specs=[af_spec, gt_spec],
        out_specs=[row_spec, row_spec, besta_spec],
        out_shape=[
            jax.ShapeDtypeStruct((B, 1, a_pad), jnp.float32),
            jax.ShapeDtypeStruct((B, 1, a_pad), jnp.float32),
            jax.ShapeDtypeStruct((B, N, 1), jnp.float32),
        ],
        scratch_shapes=[
            pltpu.VMEM((N, 1), jnp.float32),
            pltpu.VMEM((N, 1), jnp.float32),
        ],
    )(af, gt)

    scal_spec = pl.BlockSpec((1, 1), lambda b, k: (0, 0))
    s_cls, s_reg, s_np = pl.pallas_call(
        functools.partial(_loss_body, num_anchors=A, num_classes=C, ba=ba,
                          n_gt=N),
        grid=grid,
        in_specs=[
            af_spec,
            gtt_spec,
            pl.BlockSpec((1, C, ba), lambda b, k: (b, 0, k)),
            pl.BlockSpec((1, 4, ba), lambda b, k: (b, 0, k)),
            row_spec,
            row_spec,
            besta_spec,
        ],
        out_specs=[scal_spec, scal_spec, scal_spec],
        out_shape=[
            jax.ShapeDtypeStruct((1, 1), jnp.float32),
            jax.ShapeDtypeStruct((1, 1), jnp.float32),
            jax.ShapeDtypeStruct((1, 1), jnp.float32),
        ],
    )(af, gtt, cls_t, reg_t, miou, marg, besta)

    num_pos = jnp.maximum(s_np[0, 0], 1.0)
    return s_cls[0, 0] / num_pos, s_reg[0, 0] / num_pos
